# baseline (speedup 1.0000x reference)
import functools

import jax
import jax.numpy as jnp
from jax import lax
from jax.experimental import pallas as pl
from jax.experimental.pallas import tpu as pltpu
from jax.experimental.pallas import tpu_sc as plsc

P = 320000
M = 128
NC = 2
NS = 16
NW = NC * NS
RPW = P // NW
CHUNK = 400
NLANE = 16
NVEC = M // NLANE


def _sc_partials(messages):
    mesh = plsc.VectorSubcoreMesh(core_axis_name="c", subcore_axis_name="s")

    @functools.partial(
        pl.kernel,
        mesh=mesh,
        out_type=jax.ShapeDtypeStruct((NW * M,), jnp.float32),
        scratch_types=[
            pltpu.VMEM((CHUNK, M), jnp.float32),
            pltpu.VMEM((M,), jnp.float32),
        ],
    )
    def body(msg_hbm, out_hbm, buf, accv):
        wid = lax.axis_index("s") * NC + lax.axis_index("c")
        base = wid * RPW

        def chunk_body(c, accs):
            pltpu.sync_copy(msg_hbm.at[pl.ds(base + c * CHUNK, CHUNK)], buf)

            def row_body(r, a):
                return tuple(
                    a[j] + buf[r, pl.ds(j * NLANE, NLANE)] for j in range(NVEC)
                )

            return lax.fori_loop(0, CHUNK, row_body, accs)

        accs = tuple(jnp.zeros((NLANE,), jnp.float32) for _ in range(NVEC))
        accs = lax.fori_loop(0, RPW // CHUNK, chunk_body, accs)
        for j in range(NVEC):
            accv[pl.ds(j * NLANE, NLANE)] = accs[j]
        pltpu.sync_copy(accv, out_hbm.at[pl.ds(wid * M, M)])

    return body(messages).reshape((NW, M))


def _finish(part_ref, out_ref):
    out_ref[:] = jnp.sum(part_ref[:], axis=0, keepdims=True)


def kernel(messages):
    partials = _sc_partials(messages)
    out = pl.pallas_call(
        _finish,
        out_shape=jax.ShapeDtypeStruct((1, M), jnp.float32),
    )(partials)
    return out.reshape((M,))


# speedup vs baseline: 1.4930x; 1.4930x over previous
import functools

import jax
import jax.numpy as jnp
from jax import lax
from jax.experimental import pallas as pl
from jax.experimental.pallas import tpu as pltpu
from jax.experimental.pallas import tpu_sc as plsc

P = 320000
M = 128
NC = 2
NS = 16
NW = NC * NS
RPW = P // NW
CHUNK = 200
NBUF = 2
NLANE = 16
NVEC = M // NLANE


def _sc_partials(messages):
    mesh = plsc.VectorSubcoreMesh(core_axis_name="c", subcore_axis_name="s")

    @functools.partial(
        pl.kernel,
        mesh=mesh,
        out_type=jax.ShapeDtypeStruct((NW * M,), jnp.float32),
        scratch_types=[
            pltpu.VMEM((NBUF, CHUNK, M), jnp.float32),
            pltpu.VMEM((M,), jnp.float32),
        ]
        + [pltpu.SemaphoreType.DMA] * NBUF,
    )
    def body(msg_hbm, out_hbm, buf, accv, *sems):
        wid = lax.axis_index("s") * NC + lax.axis_index("c")
        base = wid * RPW
        nchunk = RPW // CHUNK
        groups = nchunk // NBUF

        def issue(c, b):
            pltpu.async_copy(
                msg_hbm.at[pl.ds(base + c * CHUNK, CHUNK)], buf.at[b], sems[b]
            )

        def wait(c, b):
            pltpu.make_async_copy(
                msg_hbm.at[pl.ds(base + c * CHUNK, CHUNK)], buf.at[b], sems[b]
            ).wait()

        for b in range(NBUF):
            issue(b, b)

        def group_body(g, accs):
            for b in range(NBUF):
                c = g * NBUF + b
                wait(c, b)

                def row_body(r, a):
                    return tuple(
                        a[j] + buf[b, r, pl.ds(j * NLANE, NLANE)]
                        for j in range(NVEC)
                    )

                accs = lax.fori_loop(0, CHUNK, row_body, accs)

                @pl.when(c + NBUF < nchunk)
                def _():
                    issue(c + NBUF, b)

            return accs

        accs = tuple(jnp.zeros((NLANE,), jnp.float32) for _ in range(NVEC))
        accs = lax.fori_loop(0, groups, group_body, accs)
        for j in range(NVEC):
            accv[pl.ds(j * NLANE, NLANE)] = accs[j]
        pltpu.sync_copy(accv, out_hbm.at[pl.ds(wid * M, M)])

    return body(messages).reshape((NW, M))


def _finish(part_ref, out_ref):
    out_ref[:] = jnp.sum(part_ref[:], axis=0, keepdims=True)


def kernel(messages):
    partials = _sc_partials(messages)
    out = pl.pallas_call(
        _finish,
        out_shape=jax.ShapeDtypeStruct((1, M), jnp.float32),
    )(partials)
    return out.reshape((M,))
